# baseline (speedup 1.0000x reference)
import functools

import jax
import jax.numpy as jnp
from jax.experimental import pallas as pl
from jax.experimental.pallas import tpu as pltpu


def _round_up(x, m):
    return (x + m - 1) // m * m


def _interp_matrix(n_in):
    n_out = 2 * n_in
    pos = jnp.arange(n_out, dtype=jnp.float32) * (n_in - 1) / (n_out - 1)
    lo = jnp.floor(pos).astype(jnp.int32)
    hi = jnp.minimum(lo + 1, n_in - 1)
    frac = (pos - lo.astype(jnp.float32)).astype(jnp.bfloat16)
    cols = jnp.arange(n_in, dtype=jnp.int32)[None, :]
    a = jnp.where(cols == lo[:, None], (1 - frac)[:, None], 0)
    a = a + jnp.where(cols == hi[:, None], frac[:, None], 0)
    return a.astype(jnp.bfloat16)


def _upsample2x_hw(x):
    ah = _interp_matrix(x.shape[-2])
    aw = _interp_matrix(x.shape[-1])
    x = jnp.einsum('ndchw,Hh->ndcHw', x, ah,
                   preferred_element_type=jnp.float32).astype(jnp.bfloat16)
    return jnp.einsum('ndchw,Ww->ndchW', x, aw,
                      preferred_element_type=jnp.float32).astype(jnp.bfloat16)


def _fused_kernel(xq_ref, w_ref, mask_ref, g_ref, b_ref, o_ref,
                  a1_scr, a2_scr, m2_scr, sc_scr, sh_scr, *,
                  d_in, d2, pb, tap_offsets, mt, cin_pad, h2, w2, wp, cnt):
    p = pl.program_id(1)
    db = pl.program_id(2)
    d0 = db * pb
    w = w_ref[...]

    def conv_planes():
        blends = []
        for j in range(pb + 2):
            du = d0 + j - 1
            duc = jnp.clip(du, 0, d2 - 1)
            num = duc * (d_in - 1)
            lo = num // (d2 - 1)
            rem = num - lo * (d2 - 1)
            frac = (rem.astype(jnp.float32) / (d2 - 1)).astype(jnp.bfloat16)
            hi = jnp.minimum(lo + 1, d_in - 1)
            vf = ((du >= 0) & (du < d2)).astype(jnp.int32).astype(jnp.float32)
            wlo = ((1.0 - frac.astype(jnp.float32)) * vf).astype(jnp.bfloat16)
            whi = (frac.astype(jnp.float32) * vf).astype(jnp.bfloat16)
            xlo = xq_ref[0, pl.ds(lo, 1), :, :][0]
            xhi = xq_ref[0, pl.ds(hi, 1), :, :][0]
            blends.append(xlo * wlo + xhi * whi)
        accs = []
        for i in range(pb):
            col = jnp.concatenate(
                [blends[i + kd][:, off:off + mt]
                 for kd in range(3) for off in tap_offsets], axis=0)
            accs.append(jnp.dot(w, col, preferred_element_type=jnp.float32))
        return accs

    @pl.when((p == 0) & (db == 0))
    def _init():
        a1_scr[...] = jnp.zeros_like(a1_scr)
        a2_scr[...] = jnp.zeros_like(a2_scr)
        m2_scr[...] = jnp.zeros_like(m2_scr)

    @pl.when(p == 0)
    def _phase0():
        a1 = jnp.zeros_like(a1_scr)
        a2 = jnp.zeros_like(a2_scr)
        m2 = jnp.zeros_like(m2_scr)
        for acc in conv_planes():
            am = jnp.where(mask_ref[...] > 0.0, acc, 0.0)
            s1 = jnp.sum(am, axis=1, keepdims=True)
            s2 = jnp.sum(am * acc, axis=1, keepdims=True)
            mu = s1 / cnt
            m2p = jnp.maximum(s2 - s1 * mu, 0.0)
            a1 = a1 + mu
            a2 = a2 + mu * mu
            m2 = m2 + m2p
        a1_scr[...] += a1
        a2_scr[...] += a2
        m2_scr[...] += m2

    @pl.when((p == 1) & (db == 0))
    def _mkscale():
        a1 = a1_scr[...]
        a2 = a2_scr[...]
        m2s = m2_scr[...]
        mean = a1 / d2
        m2 = m2s + cnt * jnp.maximum(a2 - a1 * a1 / d2, 0.0)
        var = jnp.maximum(m2 / (cnt * d2), 0.0)
        sc = g_ref[...] * jax.lax.rsqrt(var + 1e-5)
        sc_scr[...] = sc
        sh_scr[...] = b_ref[...] - mean * sc

    @pl.when(p == 1)
    def _phase1():
        sc = sc_scr[...]
        sh = sh_scr[...]
        for i, acc in enumerate(conv_planes()):
            yb = acc.astype(jnp.bfloat16)
            parts = [yb[:, h * wp:h * wp + w2] for h in range(h2)]
            yc = jnp.concatenate(parts, axis=1).astype(jnp.float32)
            v = jnp.maximum(yc * sc + sh, 0.0)
            o_ref[0, :, i] = v.reshape(v.shape[0], h2 // 2, 2 * w2)


@jax.jit
def _forward(x_ncdhw, weight, bias, gamma, beta):
    del bias

    N, Cin, D, H, W = x_ncdhw.shape
    Cout = weight.shape[0]
    D2, H2, W2 = 2 * D, 2 * H, 2 * W
    Hp, Wp = H2 + 2, W2 + 2
    hw = Hp * Wp
    mt = _round_up(hw, 128)
    hw_ext = _round_up(mt + 2 * Wp + 2, 128)
    cin_pad = _round_up(Cin, 16)

    x_t = jnp.transpose(x_ncdhw.astype(jnp.bfloat16), (0, 2, 1, 3, 4))
    xu = _upsample2x_hw(x_t)
    xq = jnp.pad(xu, ((0, 0), (0, 0), (0, cin_pad - Cin), (1, 1), (1, 1)))
    xq = xq.reshape(N, D, cin_pad, hw)
    xq = jnp.pad(xq, ((0, 0), (0, 0), (0, 0), (0, hw_ext - hw)))

    w_p = jnp.pad(weight, ((0, 0), (0, cin_pad - Cin), (0, 0), (0, 0), (0, 0)))
    w27 = jnp.transpose(w_p, (0, 2, 3, 4, 1)).reshape(Cout, 27 * cin_pad)
    w27 = w27.astype(jnp.bfloat16)

    lane = jnp.arange(mt, dtype=jnp.int32)
    mask = (((lane % Wp) < W2) & ((lane // Wp) < H2)).astype(jnp.float32)[None, :]

    tap_offsets = tuple(kh * Wp + kw for kh in range(3) for kw in range(3))
    vmem_limit = 100 * 1024 * 1024
    PB = next(pb for pb in (8, 4, 2, 1) if D2 % pb == 0)

    kern = functools.partial(
        _fused_kernel, d_in=D, d2=D2, pb=PB, tap_offsets=tap_offsets,
        mt=mt, cin_pad=cin_pad, h2=H2, w2=W2, wp=Wp, cnt=float(H2 * W2))
    flops = 2 * 2 * N * D2 * Cout * 27 * cin_pad * mt
    bytes_accessed = int(2 * N * D * cin_pad * hw_ext * 2
                         + N * Cout * D2 * H2 * W2 * 4)

    out = pl.pallas_call(
        kern,
        grid=(N, 2, D2 // PB),
        in_specs=[
            pl.BlockSpec((1, D, cin_pad, hw_ext), lambda n, p, d: (n, 0, 0, 0)),
            pl.BlockSpec((Cout, 27 * cin_pad), lambda n, p, d: (0, 0)),
            pl.BlockSpec((1, mt), lambda n, p, d: (0, 0)),
            pl.BlockSpec((Cout, 1), lambda n, p, d: (0, 0)),
            pl.BlockSpec((Cout, 1), lambda n, p, d: (0, 0)),
        ],
        out_specs=pl.BlockSpec((1, Cout, PB, H2 // 2, 2 * W2),
                               lambda n, p, d: (n, 0, p * d, 0, 0)),
        out_shape=jax.ShapeDtypeStruct((N, Cout, D2, H2 // 2, 2 * W2),
                                       jnp.float32),
        scratch_shapes=[pltpu.VMEM((Cout, 1), jnp.float32) for _ in range(5)],
        compiler_params=pltpu.CompilerParams(
            dimension_semantics=("parallel", "arbitrary", "arbitrary"),
            vmem_limit_bytes=vmem_limit),
        cost_estimate=pl.CostEstimate(
            flops=flops, transcendentals=0, bytes_accessed=bytes_accessed),
    )(xq, w27, mask, gamma.reshape(Cout, 1).astype(jnp.float32),
      beta.reshape(Cout, 1).astype(jnp.float32))

    return out.reshape(N, Cout, D2, H2, W2)


def kernel(x, weight, bias, gamma, beta):
    return _forward(x, weight, bias, gamma, beta)


# speedup vs baseline: 1.2080x; 1.2080x over previous
import functools

import jax
import jax.numpy as jnp
from jax import lax
from jax.experimental import pallas as pl
from jax.experimental.pallas import tpu as pltpu


def _round_up(x, m):
    return (x + m - 1) // m * m


def _interp_matrix(n_in):
    n_out = 2 * n_in
    pos = jnp.arange(n_out, dtype=jnp.float32) * (n_in - 1) / (n_out - 1)
    lo = jnp.floor(pos).astype(jnp.int32)
    hi = jnp.minimum(lo + 1, n_in - 1)
    frac = (pos - lo.astype(jnp.float32)).astype(jnp.bfloat16)
    cols = jnp.arange(n_in, dtype=jnp.int32)[None, :]
    a = jnp.where(cols == lo[:, None], (1 - frac)[:, None], 0)
    a = a + jnp.where(cols == hi[:, None], frac[:, None], 0)
    return a.astype(jnp.bfloat16)


def _upsample2x_hw(x):
    ah = _interp_matrix(x.shape[-2])
    aw = _interp_matrix(x.shape[-1])
    x = jnp.einsum('ndchw,Hh->ndcHw', x, ah,
                   preferred_element_type=jnp.float32).astype(jnp.bfloat16)
    return jnp.einsum('ndchw,Ww->ndchW', x, aw,
                      preferred_element_type=jnp.float32).astype(jnp.bfloat16)


def _conv_stats_kernel(xq_ref, w_ref, mask_ref, y_ref, st_ref, *,
                       d_in, d2, pb, tap_offsets, mt, cin_pad):
    d0 = pl.program_id(1) * pb

    blends = []
    for j in range(pb + 2):
        du = d0 + j - 1
        duc = jnp.clip(du, 0, d2 - 1)
        num = duc * (d_in - 1)
        lo = num // (d2 - 1)
        rem = num - lo * (d2 - 1)
        frac = (rem.astype(jnp.float32) / (d2 - 1)).astype(jnp.bfloat16)
        hi = jnp.minimum(lo + 1, d_in - 1)
        vf = ((du >= 0) & (du < d2)).astype(jnp.int32).astype(jnp.float32)
        wlo = ((1.0 - frac.astype(jnp.float32)) * vf).astype(jnp.bfloat16)
        whi = (frac.astype(jnp.float32) * vf).astype(jnp.bfloat16)
        xlo = xq_ref[0, pl.ds(lo, 1), :, :][0]
        xhi = xq_ref[0, pl.ds(hi, 1), :, :][0]
        blends.append(xlo * wlo + xhi * whi)

    w = w_ref[...]
    for p in range(pb):
        col = jnp.concatenate(
            [blends[p + kd][:, off:off + mt]
             for kd in range(3) for off in tap_offsets], axis=0)
        acc = jnp.dot(w, col, preferred_element_type=jnp.float32)
        y_ref[0, p] = acc.astype(y_ref.dtype)
        am = jnp.where(mask_ref[...] > 0.0, acc, 0.0)
        st_ref[0, p] = jnp.concatenate(
            [jnp.sum(am, axis=1, keepdims=True),
             jnp.sum(am * acc, axis=1, keepdims=True)], axis=1)


def _norm_relu_crop_kernel(y_ref, scale_ref, shift_ref, o_ref, *, pb, wp, h2, w2):
    for p in range(pb):
        yb = y_ref[0, p]
        parts = [yb[:, h * wp:h * wp + w2] for h in range(h2)]
        yc = jnp.concatenate(parts, axis=1).astype(jnp.float32)
        v = jnp.maximum(yc * scale_ref[0] + shift_ref[0], 0.0)
        o_ref[0, :, p] = v.reshape(v.shape[0], h2 // 2, 2 * w2)


@jax.jit
def _forward(x_ncdhw, weight, bias, gamma, beta):
    del bias

    N, Cin, D, H, W = x_ncdhw.shape
    Cout = weight.shape[0]
    D2, H2, W2 = 2 * D, 2 * H, 2 * W
    Hp, Wp = H2 + 2, W2 + 2
    hw = Hp * Wp
    mt = _round_up(hw, 128)
    hw_ext = _round_up(mt + 2 * Wp + 2, 128)
    cin_pad = _round_up(Cin, 16)

    x_t = jnp.transpose(x_ncdhw.astype(jnp.bfloat16), (0, 2, 1, 3, 4))
    xu = _upsample2x_hw(x_t)
    xq = jnp.pad(xu, ((0, 0), (0, 0), (0, cin_pad - Cin), (1, 1), (1, 1)))
    xq = xq.reshape(N, D, cin_pad, hw)
    xq = jnp.pad(xq, ((0, 0), (0, 0), (0, 0), (0, hw_ext - hw)))

    w_p = jnp.pad(weight, ((0, 0), (0, cin_pad - Cin), (0, 0), (0, 0), (0, 0)))
    w27 = jnp.transpose(w_p, (0, 2, 3, 4, 1)).reshape(Cout, 27 * cin_pad)
    w27 = w27.astype(jnp.bfloat16)

    lane = jnp.arange(mt, dtype=jnp.int32)
    mask = (((lane % Wp) < W2) & ((lane // Wp) < H2)).astype(jnp.float32)[None, :]

    tap_offsets = tuple(kh * Wp + kw for kh in range(3) for kw in range(3))
    vmem_limit = 100 * 1024 * 1024

    PB = 8
    kernel1 = functools.partial(_conv_stats_kernel, d_in=D, d2=D2, pb=PB,
                                tap_offsets=tap_offsets, mt=mt, cin_pad=cin_pad)
    flops = 2 * N * D2 * Cout * 27 * cin_pad * mt
    bytes_accessed = int(N * D * cin_pad * hw_ext * 2 + N * D2 * Cout * mt * 2
                         + N * D2 * Cout * 2 * 4 + mt * 4)

    conv_y, stats = pl.pallas_call(
        kernel1,
        grid=(N, D2 // PB),
        in_specs=[
            pl.BlockSpec((1, D, cin_pad, hw_ext), lambda n, d: (n, 0, 0, 0)),
            pl.BlockSpec((Cout, 27 * cin_pad), lambda n, d: (0, 0)),
            pl.BlockSpec((1, mt), lambda n, d: (0, 0)),
        ],
        out_specs=[
            pl.BlockSpec((1, PB, Cout, mt), lambda n, d: (n, d, 0, 0)),
            pl.BlockSpec((1, PB, Cout, 2), lambda n, d: (n, d, 0, 0)),
        ],
        out_shape=[
            jax.ShapeDtypeStruct((N, D2, Cout, mt), jnp.bfloat16),
            jax.ShapeDtypeStruct((N, D2, Cout, 2), jnp.float32),
        ],
        compiler_params=pltpu.CompilerParams(
            dimension_semantics=("parallel", "arbitrary"),
            vmem_limit_bytes=vmem_limit),
        cost_estimate=pl.CostEstimate(
            flops=flops, transcendentals=0, bytes_accessed=bytes_accessed),
    )(xq, w27, mask)

    cnt = float(H2 * W2)
    s1 = stats[..., 0]
    s2 = stats[..., 1]
    mu_p = s1 / cnt
    m2_p = jnp.maximum(s2 - s1 * mu_p, 0.0)
    mean = jnp.sum(s1, axis=1) / (cnt * D2)
    m2 = jnp.sum(m2_p + cnt * (mu_p - mean[:, None, :]) ** 2, axis=1)
    var = jnp.maximum(m2 / (cnt * D2), 0.0)
    scale = (gamma[None, :] * lax.rsqrt(var + 1e-5)).astype(jnp.float32)
    shift = (beta[None, :] - mean * scale).astype(jnp.float32)
    scale = scale[:, :, None]
    shift = shift[:, :, None]

    kernel2 = functools.partial(_norm_relu_crop_kernel, pb=PB, wp=Wp, h2=H2, w2=W2)
    out = pl.pallas_call(
        kernel2,
        grid=(N, D2 // PB),
        in_specs=[
            pl.BlockSpec((1, PB, Cout, mt), lambda n, d: (n, d, 0, 0)),
            pl.BlockSpec((1, Cout, 1), lambda n, d: (n, 0, 0)),
            pl.BlockSpec((1, Cout, 1), lambda n, d: (n, 0, 0)),
        ],
        out_specs=pl.BlockSpec((1, Cout, PB, H2 // 2, 2 * W2),
                               lambda n, d: (n, 0, d, 0, 0)),
        out_shape=jax.ShapeDtypeStruct((N, Cout, D2, H2 // 2, 2 * W2),
                                       jnp.float32),
        compiler_params=pltpu.CompilerParams(
            dimension_semantics=("parallel", "parallel"),
            vmem_limit_bytes=vmem_limit),
    )(conv_y, scale, shift)

    return out.reshape(N, Cout, D2, H2, W2)


def kernel(x, weight, bias, gamma, beta):
    return _forward(x, weight, bias, gamma, beta)


# speedup vs baseline: 1.3221x; 1.0945x over previous
import functools

import jax
import jax.numpy as jnp
from jax import lax
from jax.experimental import pallas as pl
from jax.experimental.pallas import tpu as pltpu


def _round_up(x, m):
    return (x + m - 1) // m * m


def _interp_matrix(n_in):
    n_out = 2 * n_in
    pos = jnp.arange(n_out, dtype=jnp.float32) * (n_in - 1) / (n_out - 1)
    lo = jnp.floor(pos).astype(jnp.int32)
    hi = jnp.minimum(lo + 1, n_in - 1)
    frac = (pos - lo.astype(jnp.float32)).astype(jnp.bfloat16)
    cols = jnp.arange(n_in, dtype=jnp.int32)[None, :]
    a = jnp.where(cols == lo[:, None], (1 - frac)[:, None], 0)
    a = a + jnp.where(cols == hi[:, None], frac[:, None], 0)
    return a.astype(jnp.bfloat16)


def _upsample2x_hw(x):
    ah = _interp_matrix(x.shape[-2])
    aw = _interp_matrix(x.shape[-1])
    x = jnp.einsum('ndchw,Hh->ndcHw', x, ah,
                   preferred_element_type=jnp.float32).astype(jnp.bfloat16)
    return jnp.einsum('ndchw,Ww->ndchW', x, aw,
                      preferred_element_type=jnp.float32).astype(jnp.bfloat16)


def _conv_stats_kernel(xq_ref, w_ref, mL_ref, mR_ref, y_ref, st_ref, *,
                       d_in, d2, pb, hwc, w2, cin_pad):
    d0 = pl.program_id(1) * pb
    pad = 2 * w2

    zp = jnp.zeros((cin_pad, pad), dtype=jnp.bfloat16)
    blends = []
    for j in range(pb + 2):
        du = d0 + j - 1
        duc = jnp.clip(du, 0, d2 - 1)
        num = duc * (d_in - 1)
        lo = num // (d2 - 1)
        rem = num - lo * (d2 - 1)
        frac = (rem.astype(jnp.float32) / (d2 - 1)).astype(jnp.bfloat16)
        hi = jnp.minimum(lo + 1, d_in - 1)
        vf = ((du >= 0) & (du < d2)).astype(jnp.int32).astype(jnp.float32)
        wlo = ((1.0 - frac.astype(jnp.float32)) * vf).astype(jnp.bfloat16)
        whi = (frac.astype(jnp.float32) * vf).astype(jnp.bfloat16)
        xlo = xq_ref[0, pl.ds(lo, 1), :, :][0]
        xhi = xq_ref[0, pl.ds(hi, 1), :, :][0]
        v0 = jnp.concatenate([zp, xlo * wlo + xhi * whi, zp], axis=1)
        blends.append((v0 * mL_ref[0], v0, v0 * mR_ref[0]))

    w = w_ref[...]
    for p in range(pb):
        col = jnp.concatenate(
            [blends[p + kd][kw][:, pad + w2 * (kh - 1) + (kw - 1):][:, :hwc]
             for kd in range(3) for kh in range(3) for kw in range(3)], axis=0)
        acc = jnp.dot(w, col, preferred_element_type=jnp.float32)
        y_ref[0, p] = acc.astype(y_ref.dtype)
        st_ref[0, p] = jnp.concatenate(
            [jnp.sum(acc, axis=1, keepdims=True),
             jnp.sum(acc * acc, axis=1, keepdims=True)], axis=1)


def _norm_relu_kernel(y_ref, scale_ref, shift_ref, o_ref, *, pb, h2, w2):
    for p in range(pb):
        yc = y_ref[0, p].astype(jnp.float32)
        v = jnp.maximum(yc * scale_ref[0] + shift_ref[0], 0.0)
        o_ref[0, :, p] = v.reshape(v.shape[0], h2 // 2, 2 * w2)


@jax.jit
def _forward(x_ncdhw, weight, bias, gamma, beta):
    del bias

    N, Cin, D, H, W = x_ncdhw.shape
    Cout = weight.shape[0]
    D2, H2, W2 = 2 * D, 2 * H, 2 * W
    hwc = H2 * W2
    ext = hwc + 4 * W2
    cin_pad = _round_up(Cin, 16)

    x_t = jnp.transpose(x_ncdhw.astype(jnp.bfloat16), (0, 2, 1, 3, 4))
    xu = _upsample2x_hw(x_t)
    xq = jnp.pad(xu, ((0, 0), (0, 0), (0, cin_pad - Cin), (0, 0), (0, 0)))
    xq = xq.reshape(N, D, cin_pad, hwc)

    w_p = jnp.pad(weight, ((0, 0), (0, cin_pad - Cin), (0, 0), (0, 0), (0, 0)))
    w27 = jnp.transpose(w_p, (0, 2, 3, 4, 1)).reshape(Cout, 27 * cin_pad)
    w27 = w27.astype(jnp.bfloat16)

    lane = jnp.arange(ext, dtype=jnp.int32)
    mL = (lane % W2 != W2 - 1).astype(jnp.bfloat16)[None, :]
    mR = (lane % W2 != 0).astype(jnp.bfloat16)[None, :]

    vmem_limit = 100 * 1024 * 1024
    PB = next(pb for pb in (8, 4, 2, 1) if D2 % pb == 0)

    kernel1 = functools.partial(_conv_stats_kernel, d_in=D, d2=D2, pb=PB,
                                hwc=hwc, w2=W2, cin_pad=cin_pad)
    flops = 2 * N * D2 * Cout * 27 * cin_pad * hwc
    bytes_accessed = int(N * D * cin_pad * hwc * 2 + N * D2 * Cout * hwc * 2
                         + N * D2 * Cout * 2 * 4)

    conv_y, stats = pl.pallas_call(
        kernel1,
        grid=(N, D2 // PB),
        in_specs=[
            pl.BlockSpec((1, D, cin_pad, hwc), lambda n, d: (n, 0, 0, 0)),
            pl.BlockSpec((Cout, 27 * cin_pad), lambda n, d: (0, 0)),
            pl.BlockSpec((1, ext), lambda n, d: (0, 0)),
            pl.BlockSpec((1, ext), lambda n, d: (0, 0)),
        ],
        out_specs=[
            pl.BlockSpec((1, PB, Cout, hwc), lambda n, d: (n, d, 0, 0)),
            pl.BlockSpec((1, PB, Cout, 2), lambda n, d: (n, d, 0, 0)),
        ],
        out_shape=[
            jax.ShapeDtypeStruct((N, D2, Cout, hwc), jnp.bfloat16),
            jax.ShapeDtypeStruct((N, D2, Cout, 2), jnp.float32),
        ],
        compiler_params=pltpu.CompilerParams(
            dimension_semantics=("parallel", "arbitrary"),
            vmem_limit_bytes=vmem_limit),
        cost_estimate=pl.CostEstimate(
            flops=flops, transcendentals=0, bytes_accessed=bytes_accessed),
    )(xq, w27, mL, mR)

    cnt = float(H2 * W2)
    s1 = stats[..., 0]
    s2 = stats[..., 1]
    mu_p = s1 / cnt
    m2_p = jnp.maximum(s2 - s1 * mu_p, 0.0)
    mean = jnp.sum(s1, axis=1) / (cnt * D2)
    m2 = jnp.sum(m2_p + cnt * (mu_p - mean[:, None, :]) ** 2, axis=1)
    var = jnp.maximum(m2 / (cnt * D2), 0.0)
    scale = (gamma[None, :] * lax.rsqrt(var + 1e-5)).astype(jnp.float32)
    shift = (beta[None, :] - mean * scale).astype(jnp.float32)
    scale = scale[:, :, None]
    shift = shift[:, :, None]

    kernel2 = functools.partial(_norm_relu_kernel, pb=PB, h2=H2, w2=W2)
    out = pl.pallas_call(
        kernel2,
        grid=(N, D2 // PB),
        in_specs=[
            pl.BlockSpec((1, PB, Cout, hwc), lambda n, d: (n, d, 0, 0)),
            pl.BlockSpec((1, Cout, 1), lambda n, d: (n, 0, 0)),
            pl.BlockSpec((1, Cout, 1), lambda n, d: (n, 0, 0)),
        ],
        out_specs=pl.BlockSpec((1, Cout, PB, H2 // 2, 2 * W2),
                               lambda n, d: (n, 0, d, 0, 0)),
        out_shape=jax.ShapeDtypeStruct((N, Cout, D2, H2 // 2, 2 * W2),
                                       jnp.float32),
        compiler_params=pltpu.CompilerParams(
            dimension_semantics=("parallel", "parallel"),
            vmem_limit_bytes=vmem_limit),
    )(conv_y, scale, shift)

    return out.reshape(N, Cout, D2, H2, W2)


def kernel(x, weight, bias, gamma, beta):
    return _forward(x, weight, bias, gamma, beta)


# speedup vs baseline: 1.3595x; 1.0283x over previous
import functools

import jax
import jax.numpy as jnp
from jax import lax
from jax.experimental import pallas as pl
from jax.experimental.pallas import tpu as pltpu


def _round_up(x, m):
    return (x + m - 1) // m * m


def _interp_matrix(n_in):
    n_out = 2 * n_in
    pos = jnp.arange(n_out, dtype=jnp.float32) * (n_in - 1) / (n_out - 1)
    lo = jnp.floor(pos).astype(jnp.int32)
    hi = jnp.minimum(lo + 1, n_in - 1)
    frac = (pos - lo.astype(jnp.float32)).astype(jnp.bfloat16)
    cols = jnp.arange(n_in, dtype=jnp.int32)[None, :]
    a = jnp.where(cols == lo[:, None], (1 - frac)[:, None], 0)
    a = a + jnp.where(cols == hi[:, None], frac[:, None], 0)
    return a.astype(jnp.bfloat16)


def _upsample2x_hw(x):
    ah = _interp_matrix(x.shape[-2])
    aw = _interp_matrix(x.shape[-1])
    x = jnp.einsum('ndchw,Hh->ndcHw', x, ah,
                   preferred_element_type=jnp.float32).astype(jnp.bfloat16)
    return jnp.einsum('ndchw,Ww->ndchW', x, aw,
                      preferred_element_type=jnp.float32).astype(jnp.bfloat16)


def _conv_stats_kernel(xq_ref, w_ref, mL_ref, mR_ref, y_ref, st_ref, *,
                       d_in, d2, pb, hwc, w2, cin_pad):
    d0 = pl.program_id(1) * pb
    pad = 2 * w2

    zp = jnp.zeros((cin_pad, pad), dtype=jnp.bfloat16)
    blends = []
    for j in range(pb + 2):
        du = d0 + j - 1
        duc = jnp.clip(du, 0, d2 - 1)
        num = duc * (d_in - 1)
        lo = num // (d2 - 1)
        rem = num - lo * (d2 - 1)
        frac = (rem.astype(jnp.float32) / (d2 - 1)).astype(jnp.bfloat16)
        hi = jnp.minimum(lo + 1, d_in - 1)
        vf = ((du >= 0) & (du < d2)).astype(jnp.int32).astype(jnp.float32)
        wlo = ((1.0 - frac.astype(jnp.float32)) * vf).astype(jnp.bfloat16)
        whi = (frac.astype(jnp.float32) * vf).astype(jnp.bfloat16)
        xlo = xq_ref[0, pl.ds(lo, 1), :, :][0]
        xhi = xq_ref[0, pl.ds(hi, 1), :, :][0]
        v0 = jnp.concatenate([zp, xlo * wlo + xhi * whi, zp], axis=1)
        blends.append((v0 * mL_ref[0], v0, v0 * mR_ref[0]))

    w = w_ref[...]
    for p in range(pb):
        col = jnp.concatenate(
            [blends[p + kd][kw][:, pad + w2 * (kh - 1) + (kw - 1):][:, :hwc]
             for kd in range(3) for kh in range(3) for kw in range(3)], axis=0)
        acc = jnp.dot(w, col, preferred_element_type=jnp.float32)
        y_ref[0, p] = acc.astype(y_ref.dtype)
        st_ref[0, p] = jnp.concatenate(
            [jnp.sum(acc, axis=1, keepdims=True),
             jnp.sum(acc * acc, axis=1, keepdims=True)], axis=1)


def _norm_relu_kernel(y_ref, scale_ref, shift_ref, o_ref, *, pb, h2, w2):
    for p in range(pb):
        yc = y_ref[0, p].astype(jnp.float32)
        v = jnp.maximum(yc * scale_ref[0] + shift_ref[0], 0.0)
        o_ref[0, :, p] = v.reshape(v.shape[0], h2 // 2, 2 * w2)


@jax.jit
def _forward(x_ncdhw, weight, bias, gamma, beta):
    del bias

    N, Cin, D, H, W = x_ncdhw.shape
    Cout = weight.shape[0]
    D2, H2, W2 = 2 * D, 2 * H, 2 * W
    hwc = H2 * W2
    ext = hwc + 4 * W2
    cin_pad = _round_up(Cin, 16)

    x_t = jnp.transpose(x_ncdhw.astype(jnp.bfloat16), (0, 2, 1, 3, 4))
    xu = _upsample2x_hw(x_t)
    xq = jnp.pad(xu, ((0, 0), (0, 0), (0, cin_pad - Cin), (0, 0), (0, 0)))
    xq = xq.reshape(N, D, cin_pad, hwc)

    w_p = jnp.pad(weight, ((0, 0), (0, cin_pad - Cin), (0, 0), (0, 0), (0, 0)))
    w27 = jnp.transpose(w_p, (0, 2, 3, 4, 1)).reshape(Cout, 27 * cin_pad)
    w27 = w27.astype(jnp.bfloat16)

    lane = jnp.arange(ext, dtype=jnp.int32)
    mL = (lane % W2 != W2 - 1).astype(jnp.bfloat16)[None, :]
    mR = (lane % W2 != 0).astype(jnp.bfloat16)[None, :]

    vmem_limit = 100 * 1024 * 1024
    PB = next(pb for pb in (16, 8, 4, 2, 1) if D2 % pb == 0)

    kernel1 = functools.partial(_conv_stats_kernel, d_in=D, d2=D2, pb=PB,
                                hwc=hwc, w2=W2, cin_pad=cin_pad)
    flops = 2 * N * D2 * Cout * 27 * cin_pad * hwc
    bytes_accessed = int(N * D * cin_pad * hwc * 2 + N * D2 * Cout * hwc * 2
                         + N * D2 * Cout * 2 * 4)

    conv_y, stats = pl.pallas_call(
        kernel1,
        grid=(N, D2 // PB),
        in_specs=[
            pl.BlockSpec((1, D, cin_pad, hwc), lambda n, d: (n, 0, 0, 0)),
            pl.BlockSpec((Cout, 27 * cin_pad), lambda n, d: (0, 0)),
            pl.BlockSpec((1, ext), lambda n, d: (0, 0)),
            pl.BlockSpec((1, ext), lambda n, d: (0, 0)),
        ],
        out_specs=[
            pl.BlockSpec((1, PB, Cout, hwc), lambda n, d: (n, d, 0, 0)),
            pl.BlockSpec((1, PB, Cout, 2), lambda n, d: (n, d, 0, 0)),
        ],
        out_shape=[
            jax.ShapeDtypeStruct((N, D2, Cout, hwc), jnp.bfloat16),
            jax.ShapeDtypeStruct((N, D2, Cout, 2), jnp.float32),
        ],
        compiler_params=pltpu.CompilerParams(
            dimension_semantics=("parallel", "arbitrary"),
            vmem_limit_bytes=vmem_limit),
        cost_estimate=pl.CostEstimate(
            flops=flops, transcendentals=0, bytes_accessed=bytes_accessed),
    )(xq, w27, mL, mR)

    cnt = float(H2 * W2)
    s1 = stats[..., 0]
    s2 = stats[..., 1]
    mu_p = s1 / cnt
    m2_p = jnp.maximum(s2 - s1 * mu_p, 0.0)
    mean = jnp.sum(s1, axis=1) / (cnt * D2)
    m2 = jnp.sum(m2_p + cnt * (mu_p - mean[:, None, :]) ** 2, axis=1)
    var = jnp.maximum(m2 / (cnt * D2), 0.0)
    scale = (gamma[None, :] * lax.rsqrt(var + 1e-5)).astype(jnp.float32)
    shift = (beta[None, :] - mean * scale).astype(jnp.float32)
    scale = scale[:, :, None]
    shift = shift[:, :, None]

    kernel2 = functools.partial(_norm_relu_kernel, pb=PB, h2=H2, w2=W2)
    out = pl.pallas_call(
        kernel2,
        grid=(N, D2 // PB),
        in_specs=[
            pl.BlockSpec((1, PB, Cout, hwc), lambda n, d: (n, d, 0, 0)),
            pl.BlockSpec((1, Cout, 1), lambda n, d: (n, 0, 0)),
            pl.BlockSpec((1, Cout, 1), lambda n, d: (n, 0, 0)),
        ],
        out_specs=pl.BlockSpec((1, Cout, PB, H2 // 2, 2 * W2),
                               lambda n, d: (n, 0, d, 0, 0)),
        out_shape=jax.ShapeDtypeStruct((N, Cout, D2, H2 // 2, 2 * W2),
                                       jnp.float32),
        compiler_params=pltpu.CompilerParams(
            dimension_semantics=("parallel", "parallel"),
            vmem_limit_bytes=vmem_limit),
    )(conv_y, scale, shift)

    return out.reshape(N, Cout, D2, H2, W2)


def kernel(x, weight, bias, gamma, beta):
    return _forward(x, weight, bias, gamma, beta)
